# baseline (speedup 1.0000x reference)
import functools

import jax
import jax.numpy as jnp
from jax import lax
from jax.experimental import pallas as pl
from jax.experimental.pallas import tpu as pltpu
from jax.experimental.pallas import tpu_sc as plsc
import numpy as np

N = 10000
E = 320000
D = 128
G = 64

NC = 2
NS = 16
LANES = 16

CHUNK = 128
CPT = 2 * -(-E // (NC * NS * CHUNK * 2))
NCHK = CPT * NC * NS
EP = NCHK * CHUNK
DEGB = 8
AGB = 8
PAD_SPREAD = 64

NP = -(-N // (NS * LANES)) * (NS * LANES)
RPT = NP // NS
ZBR = 32

_LOG2 = float(np.log(2.0))


def _deg_body(ed, out, blk0, blk1, onesv, degv, zs,
              semi0, semi1, sems, acc):
    cid = lax.axis_index("c")
    sid = lax.axis_index("s")
    ones16 = jnp.ones((LANES,), jnp.float32)
    zeros16 = jnp.zeros((LANES,), jnp.float32)
    for k in range(CHUNK // LANES):
        onesv[pl.ds(k * LANES, LANES)] = ones16
    for k in range(RPT // LANES):
        zs[pl.ds(k * LANES, LANES)] = zeros16
    my = pl.ds(sid * RPT, RPT)
    pltpu.sync_copy(zs, acc.at[my])
    plsc.subcore_barrier()
    base = cid * (NS * CPT) + sid * CPT
    nblk = CPT // DEGB
    blks = (blk0, blk1)
    semis = (semi0, semi1)
    for b in range(3):
        row0 = b * NCHK + base
        pltpu.async_copy(ed.at[pl.ds(row0, DEGB)], blk0, semi0).wait()
        for i in range(nblk):
            s_ = i % 2
            o_ = 1 - s_
            if i + 1 < nblk:
                nxt = pltpu.async_copy(
                    ed.at[pl.ds(row0 + (i + 1) * DEGB, DEGB)], blks[o_],
                    semis[o_])
            scat = [pltpu.async_copy(onesv, acc.at[blks[s_].at[k, 1]], sems,
                                     add=True)
                    for k in range(DEGB)]
            for d in scat:
                d.wait()
            if i + 1 < nblk:
                nxt.wait()
        plsc.subcore_barrier()
        pltpu.sync_copy(acc.at[my], degv)
        pltpu.sync_copy(degv, out.at[b * NC + cid, my])
        pltpu.sync_copy(zs, acc.at[my])
        plsc.subcore_barrier()


def _run_deg(ed):
    mesh = plsc.VectorSubcoreMesh(core_axis_name="c", subcore_axis_name="s")
    shp = jax.ShapeDtypeStruct((3 * NC, NP), jnp.float32)
    fn = pl.kernel(
        _deg_body,
        out_type=shp,
        mesh=mesh,
        scratch_types=[
            pltpu.VMEM((DEGB, 2, CHUNK), jnp.int32),
            pltpu.VMEM((DEGB, 2, CHUNK), jnp.int32),
            pltpu.VMEM((CHUNK,), jnp.float32),
            pltpu.VMEM((RPT,), jnp.float32),
            pltpu.VMEM((RPT,), jnp.float32),
            pltpu.SemaphoreType.DMA,
            pltpu.SemaphoreType.DMA,
            pltpu.SemaphoreType.DMA,
            pltpu.VMEM_SHARED((NP,), jnp.float32),
        ],
    )
    return fn(ed)


def _agg_body(ed, y1, y2, y3, out, blk0, blk1, rows0, rows1, zb, acc,
              semi0, semi1, semg0, semg1, sems0, sems1):
    cid = lax.axis_index("c")
    sid = lax.axis_index("s")
    zeros16 = jnp.zeros((LANES,), jnp.float32)

    def zrow(r, carry):
        for k in range(D // LANES):
            zb[r, pl.ds(k * LANES, LANES)] = zeros16
        return carry
    lax.fori_loop(0, ZBR, zrow, 0)
    for q in range(RPT // ZBR):
        pltpu.sync_copy(zb, acc.at[pl.ds(sid * RPT + q * ZBR, ZBR)])
    plsc.subcore_barrier()

    blks = (blk0, blk1)
    rows = (rows0, rows1)
    semi = (semi0, semi1)
    semg = (semg0, semg1)
    sems = (sems0, sems1)
    base = cid * (NS * CPT) + sid * CPT
    nblk = CPT // AGB

    def bfetch(row, u):
        return pltpu.async_copy(ed.at[pl.ds(row, AGB)], blks[u], semi[u])

    def wait_bfetch(u):
        pltpu.make_async_copy(ed.at[pl.ds(0, AGB)], blks[u], semi[u]).wait()

    def gather(y, u, k, sl):
        return pltpu.async_copy(y.at[blks[u].at[k, 0]], rows[sl], semg[sl])

    def scatter(u, k, sl):
        return pltpu.async_copy(rows[sl], acc.at[blks[u].at[k, 1]], sems[sl],
                                add=True)

    def wait_scatter(y, sl):
        pltpu.make_async_copy(y.at[pl.ds(0, CHUNK)], rows[sl], sems[sl]).wait()

    def block(y, u, pref_row, first):
        wait_bfetch(u)
        if not first:
            wait_scatter(y, 0)
        g = gather(y, u, 0, 0)
        if not first:
            wait_scatter(y, 1)
        bfetch(pref_row, 1 - u)
        g.wait()
        scatter(u, 0, 0)
        g = gather(y, u, 1, 1)
        g.wait()
        scatter(u, 1, 1)
        for k in range(2, AGB):
            sl = k % 2
            wait_scatter(y, sl)
            g = gather(y, u, k, sl)
            g.wait()
            scatter(u, k, sl)

    for b, y in enumerate((y1, y2, y3)):
        row0 = b * NCHK + base
        bfetch(row0, 0)
        block(y, 0, row0 + AGB, True)
        block(y, 1, row0 + 2 * AGB, False)

        cap = row0 + (nblk - 2) * AGB
        def dbl(d, carry):
            r2 = row0 + 2 * d * AGB
            block(y, 0, r2 + AGB, False)
            block(y, 1, jnp.minimum(r2 + 2 * AGB, cap), False)
            return carry
        lax.fori_loop(1, nblk // 2, dbl, 0)
        wait_bfetch(0)
        wait_scatter(y, 0)
        wait_scatter(y, 1)
        plsc.subcore_barrier()
        for q in range(RPT // CHUNK):
            sl = pl.ds(sid * RPT + q * CHUNK, CHUNK)
            pltpu.sync_copy(acc.at[sl], rows0)
            pltpu.sync_copy(rows0, out.at[b * NC + cid, sl])
            for w in range(CHUNK // ZBR):
                pltpu.sync_copy(
                    zb, acc.at[pl.ds(sid * RPT + q * CHUNK + w * ZBR, ZBR)])
        plsc.subcore_barrier()


def _run_agg(ed, y1, y2, y3):
    mesh = plsc.VectorSubcoreMesh(core_axis_name="c", subcore_axis_name="s")
    shp = jax.ShapeDtypeStruct((3 * NC, NP, D), jnp.float32)
    fn = pl.kernel(
        _agg_body,
        out_type=shp,
        mesh=mesh,
        scratch_types=[
            pltpu.VMEM((AGB, 2, CHUNK), jnp.int32),
            pltpu.VMEM((AGB, 2, CHUNK), jnp.int32),
            pltpu.VMEM((CHUNK, D), jnp.float32),
            pltpu.VMEM((CHUNK, D), jnp.float32),
            pltpu.VMEM((ZBR, D), jnp.float32),
            pltpu.VMEM_SHARED((NP, D), jnp.float32),
            pltpu.SemaphoreType.DMA,
            pltpu.SemaphoreType.DMA,
            pltpu.SemaphoreType.DMA,
            pltpu.SemaphoreType.DMA,
            pltpu.SemaphoreType.DMA,
            pltpu.SemaphoreType.DMA,
        ],
    )
    return fn(ed, y1, y2, y3)


def _scale_body(x1_ref, x2_ref, x3_ref, dp_ref, y1_ref, y2_ref, y3_ref):
    for b, (x_ref, y_ref) in enumerate(
            ((x1_ref, y1_ref), (x2_ref, y2_ref), (x3_ref, y3_ref))):
        deg = dp_ref[2 * b] + dp_ref[2 * b + 1]
        r = lax.rsqrt(jnp.maximum(deg[:N], 1.0))
        y_ref[:N] = x_ref[...] * r[:, None]
        y_ref[N:] = jnp.zeros((NP - N, D), jnp.float32)


def _run_scale(x1, x2, x3, deg_part):
    yshp = jax.ShapeDtypeStruct((NP, D), jnp.float32)
    return pl.pallas_call(
        _scale_body,
        out_shape=(yshp, yshp, yshp),
        compiler_params=pltpu.CompilerParams(vmem_limit_bytes=100 * 1024 * 1024),
    )(x1, x2, x3, deg_part)


def _mlp_apply(x, w_ref, b_ref, a0, a1, a2):
    h = jnp.dot(x, w_ref[0], preferred_element_type=jnp.float32) + b_ref[0]
    h = jnp.where(h >= 0, h, a0 * h)
    h = jnp.dot(h, w_ref[1], preferred_element_type=jnp.float32) + b_ref[1]
    h = jnp.where(h >= 0, h, a1 * h)
    h = jnp.dot(h, w_ref[2], preferred_element_type=jnp.float32) + b_ref[2]
    h = jnp.where(h >= 0, h, a2 * h)
    sc = jnp.dot(x, w_ref[3], preferred_element_type=jnp.float32) + b_ref[3]
    return h + sc


def _loss_body(tp_ref, dp_ref, ew_ref, eb_ref, batch_ref,
               lw_ref, lb_ref, gw_ref, gb_ref, al_ref, out_ref):
    t = tp_ref[0, 0] + tp_ref[0, 1]
    deg = dp_ref[0, 0] + dp_ref[0, 1]
    r = lax.rsqrt(jnp.maximum(deg, 1.0))
    agg = t * r[:, None]
    z = jnp.dot(agg, ew_ref[0], preferred_element_type=jnp.float32)
    z = jnp.maximum(z + eb_ref[0, 0], 0.0)
    zn = z[:N]

    bt = batch_ref[0, 0]
    gids = lax.broadcasted_iota(jnp.int32, (1, G), 1)
    pos = (bt[:, None] == gids).astype(jnp.float32)
    g = jax.lax.dot_general(pos, zn, (((0,), (0,)), ((), ())),
                            preferred_element_type=jnp.float32)

    a = al_ref[0]
    zl = _mlp_apply(zn, lw_ref, lb_ref, a[0], a[1], a[2])
    gl = _mlp_apply(g, gw_ref, gb_ref, a[3], a[4], a[5])

    res = jax.lax.dot_general(zl, gl, (((1,), (1,)), ((), ())),
                              preferred_element_type=jnp.float32)
    neg_res = -res
    sp = jnp.maximum(neg_res, 0.0) + jnp.log1p(jnp.exp(-jnp.abs(neg_res)))
    e_pos = _LOG2 - sp
    e_neg = sp + res - _LOG2
    npos = jnp.sum(pos)
    nneg = float(N * G) - npos
    e_pos_sum = jnp.sum(e_pos * pos)
    e_neg_sum = jnp.sum(e_neg) - jnp.sum(e_neg * pos)
    loss = 2.0 * (e_neg_sum / jnp.maximum(nneg, 1.0)
                  - e_pos_sum / jnp.maximum(npos, 1.0))
    out_ref[0] = jnp.full((8, 128), loss, jnp.float32)


def _run_loss(t_part, deg_part, enc_w, enc_b, batch, lw, lb, gw, gb, alphas):
    full = lambda *shape: pl.BlockSpec(shape, lambda b: (0,) * len(shape))
    return pl.pallas_call(
        _loss_body,
        grid=(3,),
        in_specs=[
            pl.BlockSpec((1, NC, NP, D), lambda b: (b, 0, 0, 0)),
            pl.BlockSpec((1, NC, NP), lambda b: (b, 0, 0)),
            pl.BlockSpec((1, D, D), lambda b: (b, 0, 0)),
            pl.BlockSpec((1, 1, D), lambda b: (b, 0, 0)),
            pl.BlockSpec((1, 1, N), lambda b: (b, 0, 0)),
            full(4, D, D),
            full(4, D),
            full(4, D, D),
            full(4, D),
            full(1, 8),
        ],
        out_specs=pl.BlockSpec((1, 8, 128), lambda b: (b, 0, 0)),
        out_shape=jax.ShapeDtypeStruct((3, 8, 128), jnp.float32),
        compiler_params=pltpu.CompilerParams(vmem_limit_bytes=100 * 1024 * 1024),
    )(t_part, deg_part, enc_w, enc_b, batch, lw, lb, gw, gb, alphas)


def kernel(x1, x2, x3, edge_index1, edge_index2, edge_index3,
           batch1, batch2, batch3, params):
    pad_fill = (N + (jnp.arange(EP - E, dtype=jnp.int32) % PAD_SPREAD))

    def pad_idx(ei):
        sr = jnp.concatenate([ei[0], pad_fill]).reshape(NCHK, 1, CHUNK)
        ds_ = jnp.concatenate([ei[1], pad_fill]).reshape(NCHK, 1, CHUNK)
        return jnp.concatenate([sr, ds_], axis=1)

    ed = jnp.concatenate([pad_idx(edge_index1), pad_idx(edge_index2),
                          pad_idx(edge_index3)], axis=0)

    deg_flat = _run_deg(ed)
    deg_part = deg_flat.reshape(3, NC, NP)

    y1, y2, y3 = _run_scale(x1, x2, x3, deg_flat)

    t_flat = _run_agg(ed, y1, y2, y3)
    t_part = t_flat.reshape(3, NC, NP, D)

    p = params
    enc_w = jnp.stack([p["enc1"]["W"], p["enc2"]["W"], p["enc3"]["W"]])
    enc_b = jnp.stack([p["enc1"]["b"], p["enc2"]["b"],
                       p["enc3"]["b"]])[:, None, :]
    batch = jnp.stack([batch1, batch2, batch3])[:, None, :]

    def mlp_params(mp):
        w = jnp.stack([mp["fc1"]["W"], mp["fc2"]["W"], mp["fc3"]["W"],
                       mp["sc"]["W"]])
        b = jnp.stack([mp["fc1"]["b"], mp["fc2"]["b"], mp["fc3"]["b"],
                       mp["sc"]["b"]])
        return w, b

    lw, lb = mlp_params(p["local"])
    gw, gb = mlp_params(p["global"])
    alphas = jnp.concatenate([p["local"]["a1"], p["local"]["a2"],
                              p["local"]["a3"], p["global"]["a1"],
                              p["global"]["a2"], p["global"]["a3"],
                              jnp.zeros((2,), jnp.float32)])[None, :]

    losses = _run_loss(t_part, deg_part, enc_w, enc_b, batch,
                       lw, lb, gw, gb, alphas)
    return losses[0, 0, 0] + losses[1, 0, 0] + losses[2, 0, 0]


# speedup vs baseline: 1.1333x; 1.1333x over previous
import functools

import jax
import jax.numpy as jnp
from jax import lax
from jax.experimental import pallas as pl
from jax.experimental.pallas import tpu as pltpu
from jax.experimental.pallas import tpu_sc as plsc
import numpy as np

N = 10000
E = 320000
D = 128
G = 64

NC = 2
NS = 16
LANES = 16

CHUNK = 128
CPT = 2 * -(-E // (NC * NS * CHUNK * 2))
NCHK = CPT * NC * NS
EP = NCHK * CHUNK
DEGB = 8
AGB = 8
PAD_SPREAD = 64

NP = -(-N // (NS * LANES)) * (NS * LANES)
RPT = NP // NS
ZBR = 32

_LOG2 = float(np.log(2.0))


def _deg_body(ed, out, blk0, blk1, onesv, degv, zs,
              semi0, semi1, sems, acc):
    cid = lax.axis_index("c")
    sid = lax.axis_index("s")
    ones16 = jnp.ones((LANES,), jnp.float32)
    zeros16 = jnp.zeros((LANES,), jnp.float32)
    for k in range(CHUNK // LANES):
        onesv[pl.ds(k * LANES, LANES)] = ones16
    for k in range(RPT // LANES):
        zs[pl.ds(k * LANES, LANES)] = zeros16
    my = pl.ds(sid * RPT, RPT)
    pltpu.sync_copy(zs, acc.at[my])
    plsc.subcore_barrier()
    base = cid * (NS * CPT) + sid * CPT
    nblk = CPT // DEGB
    blks = (blk0, blk1)
    semis = (semi0, semi1)
    for b in range(3):
        row0 = b * NCHK + base
        pltpu.async_copy(ed.at[pl.ds(row0, DEGB)], blk0, semi0).wait()
        for i in range(nblk):
            s_ = i % 2
            o_ = 1 - s_
            if i + 1 < nblk:
                nxt = pltpu.async_copy(
                    ed.at[pl.ds(row0 + (i + 1) * DEGB, DEGB)], blks[o_],
                    semis[o_])
            scat = [pltpu.async_copy(onesv, acc.at[blks[s_].at[k, 1]], sems,
                                     add=True)
                    for k in range(DEGB)]
            for d in scat:
                d.wait()
            if i + 1 < nblk:
                nxt.wait()
        plsc.subcore_barrier()
        pltpu.sync_copy(acc.at[my], degv)
        pltpu.sync_copy(degv, out.at[b * NC + cid, my])
        pltpu.sync_copy(zs, acc.at[my])
        plsc.subcore_barrier()


def _run_deg(ed):
    mesh = plsc.VectorSubcoreMesh(core_axis_name="c", subcore_axis_name="s")
    shp = jax.ShapeDtypeStruct((3 * NC, NP), jnp.float32)
    fn = pl.kernel(
        _deg_body,
        out_type=shp,
        mesh=mesh,
        scratch_types=[
            pltpu.VMEM((DEGB, 2, CHUNK), jnp.int32),
            pltpu.VMEM((DEGB, 2, CHUNK), jnp.int32),
            pltpu.VMEM((CHUNK,), jnp.float32),
            pltpu.VMEM((RPT,), jnp.float32),
            pltpu.VMEM((RPT,), jnp.float32),
            pltpu.SemaphoreType.DMA,
            pltpu.SemaphoreType.DMA,
            pltpu.SemaphoreType.DMA,
            pltpu.VMEM_SHARED((NP,), jnp.float32),
        ],
    )
    return fn(ed)


def _agg_body(ed, y1, y2, y3, out, blk0, blk1, rows0, rows1, zb, acc,
              semi0, semi1, semg0, semg1, sems0, sems1):
    cid = lax.axis_index("c")
    sid = lax.axis_index("s")
    zeros16 = jnp.zeros((LANES,), jnp.float32)

    def zrow(r, carry):
        for k in range(D // LANES):
            zb[r, pl.ds(k * LANES, LANES)] = zeros16
        return carry
    lax.fori_loop(0, ZBR, zrow, 0)
    for q in range(RPT // ZBR):
        pltpu.sync_copy(zb, acc.at[pl.ds(sid * RPT + q * ZBR, ZBR)])
    plsc.subcore_barrier()

    blks = (blk0, blk1)
    rows = (rows0, rows1)
    semi = (semi0, semi1)
    semg = (semg0, semg1)
    sems = (sems0, sems1)
    base = cid * (NS * CPT) + sid * CPT
    nblk = CPT // AGB

    def bfetch(row, u):
        return pltpu.async_copy(ed.at[pl.ds(row, AGB)], blks[u], semi[u])

    def wait_bfetch(u):
        pltpu.make_async_copy(ed.at[pl.ds(0, AGB)], blks[u], semi[u]).wait()

    def gather(y, u, k, sl):
        return pltpu.async_copy(y.at[blks[u].at[k, 0]], rows[sl], semg[sl])

    def scatter(u, k, sl):
        return pltpu.async_copy(rows[sl], acc.at[blks[u].at[k, 1]], sems[sl],
                                add=True)

    def wait_scatter(y, sl):
        pltpu.make_async_copy(y.at[pl.ds(0, CHUNK)], rows[sl], sems[sl]).wait()

    def wait_gather(y, sl):
        pltpu.make_async_copy(y.at[pl.ds(0, CHUNK)], rows[sl], semg[sl]).wait()

    def block(y, u, pref_row, first):
        wait_bfetch(u)
        for k in range(AGB):
            sl = k % 2
            if not (first and k < 2):
                wait_scatter(y, sl)
            gather(y, u, k, sl)
            if not (first and k == 0):
                pu, pk = (u, k - 1) if k >= 1 else (1 - u, AGB - 1)
                wait_gather(y, 1 - sl)
                scatter(pu, pk, 1 - sl)
            if k == 1:
                bfetch(pref_row, 1 - u)

    for b, y in enumerate((y1, y2, y3)):
        row0 = b * NCHK + base
        bfetch(row0, 0)
        block(y, 0, row0 + AGB, True)
        block(y, 1, row0 + 2 * AGB, False)

        cap = row0 + (nblk - 2) * AGB
        def dbl(d, carry):
            r2 = row0 + 2 * d * AGB
            block(y, 0, r2 + AGB, False)
            block(y, 1, jnp.minimum(r2 + 2 * AGB, cap), False)
            return carry
        lax.fori_loop(1, nblk // 2, dbl, 0)
        wait_bfetch(0)
        wait_gather(y, 1)
        scatter(1, AGB - 1, 1)
        wait_scatter(y, 0)
        wait_scatter(y, 1)
        plsc.subcore_barrier()
        for q in range(RPT // CHUNK):
            sl = pl.ds(sid * RPT + q * CHUNK, CHUNK)
            pltpu.sync_copy(acc.at[sl], rows0)
            pltpu.sync_copy(rows0, out.at[b * NC + cid, sl])
            for w in range(CHUNK // ZBR):
                pltpu.sync_copy(
                    zb, acc.at[pl.ds(sid * RPT + q * CHUNK + w * ZBR, ZBR)])
        plsc.subcore_barrier()


def _run_agg(ed, y1, y2, y3):
    mesh = plsc.VectorSubcoreMesh(core_axis_name="c", subcore_axis_name="s")
    shp = jax.ShapeDtypeStruct((3 * NC, NP, D), jnp.float32)
    fn = pl.kernel(
        _agg_body,
        out_type=shp,
        mesh=mesh,
        scratch_types=[
            pltpu.VMEM((AGB, 2, CHUNK), jnp.int32),
            pltpu.VMEM((AGB, 2, CHUNK), jnp.int32),
            pltpu.VMEM((CHUNK, D), jnp.float32),
            pltpu.VMEM((CHUNK, D), jnp.float32),
            pltpu.VMEM((ZBR, D), jnp.float32),
            pltpu.VMEM_SHARED((NP, D), jnp.float32),
            pltpu.SemaphoreType.DMA,
            pltpu.SemaphoreType.DMA,
            pltpu.SemaphoreType.DMA,
            pltpu.SemaphoreType.DMA,
            pltpu.SemaphoreType.DMA,
            pltpu.SemaphoreType.DMA,
        ],
    )
    return fn(ed, y1, y2, y3)


def _scale_body(x1_ref, x2_ref, x3_ref, dp_ref, y1_ref, y2_ref, y3_ref):
    for b, (x_ref, y_ref) in enumerate(
            ((x1_ref, y1_ref), (x2_ref, y2_ref), (x3_ref, y3_ref))):
        deg = dp_ref[2 * b] + dp_ref[2 * b + 1]
        r = lax.rsqrt(jnp.maximum(deg[:N], 1.0))
        y_ref[:N] = x_ref[...] * r[:, None]
        y_ref[N:] = jnp.zeros((NP - N, D), jnp.float32)


def _run_scale(x1, x2, x3, deg_part):
    yshp = jax.ShapeDtypeStruct((NP, D), jnp.float32)
    return pl.pallas_call(
        _scale_body,
        out_shape=(yshp, yshp, yshp),
        compiler_params=pltpu.CompilerParams(vmem_limit_bytes=100 * 1024 * 1024),
    )(x1, x2, x3, deg_part)


def _mlp_apply(x, w_ref, b_ref, a0, a1, a2):
    h = jnp.dot(x, w_ref[0], preferred_element_type=jnp.float32) + b_ref[0]
    h = jnp.where(h >= 0, h, a0 * h)
    h = jnp.dot(h, w_ref[1], preferred_element_type=jnp.float32) + b_ref[1]
    h = jnp.where(h >= 0, h, a1 * h)
    h = jnp.dot(h, w_ref[2], preferred_element_type=jnp.float32) + b_ref[2]
    h = jnp.where(h >= 0, h, a2 * h)
    sc = jnp.dot(x, w_ref[3], preferred_element_type=jnp.float32) + b_ref[3]
    return h + sc


def _loss_body(tp_ref, dp_ref, ew_ref, eb_ref, batch_ref,
               lw_ref, lb_ref, gw_ref, gb_ref, al_ref, out_ref):
    t = tp_ref[0, 0] + tp_ref[0, 1]
    deg = dp_ref[0, 0] + dp_ref[0, 1]
    r = lax.rsqrt(jnp.maximum(deg, 1.0))
    agg = t * r[:, None]
    z = jnp.dot(agg, ew_ref[0], preferred_element_type=jnp.float32)
    z = jnp.maximum(z + eb_ref[0, 0], 0.0)
    zn = z[:N]

    bt = batch_ref[0, 0]
    gids = lax.broadcasted_iota(jnp.int32, (1, G), 1)
    pos = (bt[:, None] == gids).astype(jnp.float32)
    g = jax.lax.dot_general(pos, zn, (((0,), (0,)), ((), ())),
                            preferred_element_type=jnp.float32)

    a = al_ref[0]
    zl = _mlp_apply(zn, lw_ref, lb_ref, a[0], a[1], a[2])
    gl = _mlp_apply(g, gw_ref, gb_ref, a[3], a[4], a[5])

    res = jax.lax.dot_general(zl, gl, (((1,), (1,)), ((), ())),
                              preferred_element_type=jnp.float32)
    neg_res = -res
    sp = jnp.maximum(neg_res, 0.0) + jnp.log1p(jnp.exp(-jnp.abs(neg_res)))
    e_pos = _LOG2 - sp
    e_neg = sp + res - _LOG2
    npos = jnp.sum(pos)
    nneg = float(N * G) - npos
    e_pos_sum = jnp.sum(e_pos * pos)
    e_neg_sum = jnp.sum(e_neg) - jnp.sum(e_neg * pos)
    loss = 2.0 * (e_neg_sum / jnp.maximum(nneg, 1.0)
                  - e_pos_sum / jnp.maximum(npos, 1.0))
    out_ref[0] = jnp.full((8, 128), loss, jnp.float32)


def _run_loss(t_part, deg_part, enc_w, enc_b, batch, lw, lb, gw, gb, alphas):
    full = lambda *shape: pl.BlockSpec(shape, lambda b: (0,) * len(shape))
    return pl.pallas_call(
        _loss_body,
        grid=(3,),
        in_specs=[
            pl.BlockSpec((1, NC, NP, D), lambda b: (b, 0, 0, 0)),
            pl.BlockSpec((1, NC, NP), lambda b: (b, 0, 0)),
            pl.BlockSpec((1, D, D), lambda b: (b, 0, 0)),
            pl.BlockSpec((1, 1, D), lambda b: (b, 0, 0)),
            pl.BlockSpec((1, 1, N), lambda b: (b, 0, 0)),
            full(4, D, D),
            full(4, D),
            full(4, D, D),
            full(4, D),
            full(1, 8),
        ],
        out_specs=pl.BlockSpec((1, 8, 128), lambda b: (b, 0, 0)),
        out_shape=jax.ShapeDtypeStruct((3, 8, 128), jnp.float32),
        compiler_params=pltpu.CompilerParams(vmem_limit_bytes=100 * 1024 * 1024),
    )(t_part, deg_part, enc_w, enc_b, batch, lw, lb, gw, gb, alphas)


def kernel(x1, x2, x3, edge_index1, edge_index2, edge_index3,
           batch1, batch2, batch3, params):
    pad_fill = (N + (jnp.arange(EP - E, dtype=jnp.int32) % PAD_SPREAD))

    def pad_idx(ei):
        sr = jnp.concatenate([ei[0], pad_fill]).reshape(NCHK, 1, CHUNK)
        ds_ = jnp.concatenate([ei[1], pad_fill]).reshape(NCHK, 1, CHUNK)
        return jnp.concatenate([sr, ds_], axis=1)

    ed = jnp.concatenate([pad_idx(edge_index1), pad_idx(edge_index2),
                          pad_idx(edge_index3)], axis=0)

    deg_flat = _run_deg(ed)
    deg_part = deg_flat.reshape(3, NC, NP)

    y1, y2, y3 = _run_scale(x1, x2, x3, deg_flat)

    t_flat = _run_agg(ed, y1, y2, y3)
    t_part = t_flat.reshape(3, NC, NP, D)

    p = params
    enc_w = jnp.stack([p["enc1"]["W"], p["enc2"]["W"], p["enc3"]["W"]])
    enc_b = jnp.stack([p["enc1"]["b"], p["enc2"]["b"],
                       p["enc3"]["b"]])[:, None, :]
    batch = jnp.stack([batch1, batch2, batch3])[:, None, :]

    def mlp_params(mp):
        w = jnp.stack([mp["fc1"]["W"], mp["fc2"]["W"], mp["fc3"]["W"],
                       mp["sc"]["W"]])
        b = jnp.stack([mp["fc1"]["b"], mp["fc2"]["b"], mp["fc3"]["b"],
                       mp["sc"]["b"]])
        return w, b

    lw, lb = mlp_params(p["local"])
    gw, gb = mlp_params(p["global"])
    alphas = jnp.concatenate([p["local"]["a1"], p["local"]["a2"],
                              p["local"]["a3"], p["global"]["a1"],
                              p["global"]["a2"], p["global"]["a3"],
                              jnp.zeros((2,), jnp.float32)])[None, :]

    losses = _run_loss(t_part, deg_part, enc_w, enc_b, batch,
                       lw, lb, gw, gb, alphas)
    return losses[0, 0, 0] + losses[1, 0, 0] + losses[2, 0, 0]


# speedup vs baseline: 1.1563x; 1.0203x over previous
import functools

import jax
import jax.numpy as jnp
from jax import lax
from jax.experimental import pallas as pl
from jax.experimental.pallas import tpu as pltpu
from jax.experimental.pallas import tpu_sc as plsc
import numpy as np

N = 10000
E = 320000
D = 128
G = 64

NC = 2
NS = 16
LANES = 16

CHUNK = 128
CPT = 2 * -(-E // (NC * NS * CHUNK * 2))
NCHK = CPT * NC * NS
EP = NCHK * CHUNK
DEGB = 8
AGB = 8
PAD_SPREAD = 64

NP = -(-N // (NS * LANES)) * (NS * LANES)
RPT = NP // NS
ZBR = 32

_LOG2 = float(np.log(2.0))


def _deg_body(ed, out, blk0, blk1, onesv, degv, zs,
              semi0, semi1, sems, acc):
    cid = lax.axis_index("c")
    sid = lax.axis_index("s")
    ones16 = jnp.ones((LANES,), jnp.float32)
    zeros16 = jnp.zeros((LANES,), jnp.float32)
    for k in range(CHUNK // LANES):
        onesv[pl.ds(k * LANES, LANES)] = ones16
    for k in range(RPT // LANES):
        zs[pl.ds(k * LANES, LANES)] = zeros16
    my = pl.ds(sid * RPT, RPT)
    pltpu.sync_copy(zs, acc.at[my])
    plsc.subcore_barrier()
    base = cid * (NS * CPT) + sid * CPT
    nblk = CPT // DEGB
    blks = (blk0, blk1)
    semis = (semi0, semi1)
    for b in range(3):
        row0 = b * NCHK + base
        pltpu.async_copy(ed.at[pl.ds(row0, DEGB)], blk0, semi0).wait()
        for i in range(nblk):
            s_ = i % 2
            o_ = 1 - s_
            if i + 1 < nblk:
                nxt = pltpu.async_copy(
                    ed.at[pl.ds(row0 + (i + 1) * DEGB, DEGB)], blks[o_],
                    semis[o_])
            scat = [pltpu.async_copy(onesv, acc.at[blks[s_].at[k, 1]], sems,
                                     add=True)
                    for k in range(DEGB)]
            for d in scat:
                d.wait()
            if i + 1 < nblk:
                nxt.wait()
        plsc.subcore_barrier()
        pltpu.sync_copy(acc.at[my], degv)
        pltpu.sync_copy(degv, out.at[b * NC + cid, my])
        pltpu.sync_copy(zs, acc.at[my])
        plsc.subcore_barrier()


def _run_deg(ed):
    mesh = plsc.VectorSubcoreMesh(core_axis_name="c", subcore_axis_name="s")
    shp = jax.ShapeDtypeStruct((3 * NC, NP), jnp.float32)
    fn = pl.kernel(
        _deg_body,
        out_type=shp,
        mesh=mesh,
        scratch_types=[
            pltpu.VMEM((DEGB, 2, CHUNK), jnp.int32),
            pltpu.VMEM((DEGB, 2, CHUNK), jnp.int32),
            pltpu.VMEM((CHUNK,), jnp.float32),
            pltpu.VMEM((RPT,), jnp.float32),
            pltpu.VMEM((RPT,), jnp.float32),
            pltpu.SemaphoreType.DMA,
            pltpu.SemaphoreType.DMA,
            pltpu.SemaphoreType.DMA,
            pltpu.VMEM_SHARED((NP,), jnp.float32),
        ],
    )
    return fn(ed)


def _agg_body(ed, y1, y2, y3, out, blk0, blk1, rows0, rows1, zb, acc,
              semi0, semi1, semg0, semg1, sems0, sems1):
    cid = lax.axis_index("c")
    sid = lax.axis_index("s")
    zeros16 = jnp.zeros((LANES,), jnp.float32)

    def zrow(r, carry):
        for k in range(D // LANES):
            zb[r, pl.ds(k * LANES, LANES)] = zeros16
        return carry
    lax.fori_loop(0, ZBR, zrow, 0)
    for q in range(RPT // ZBR):
        pltpu.sync_copy(zb, acc.at[pl.ds(sid * RPT + q * ZBR, ZBR)])
    plsc.subcore_barrier()

    blks = (blk0, blk1)
    rows = (rows0, rows1)
    semi = (semi0, semi1)
    semg = (semg0, semg1)
    sems = (sems0, sems1)
    base = cid * (NS * CPT) + sid * CPT
    nblk = CPT // AGB

    def bfetch(row, u):
        return pltpu.async_copy(ed.at[pl.ds(row, AGB)], blks[u], semi[u])

    def wait_bfetch(u):
        pltpu.make_async_copy(ed.at[pl.ds(0, AGB)], blks[u], semi[u]).wait()

    def gather(y, u, k, sl):
        return pltpu.async_copy(y.at[blks[u].at[k, 0]], rows[sl], semg[sl])

    def scatter(u, k, sl):
        return pltpu.async_copy(rows[sl], acc.at[blks[u].at[k, 1]], sems[sl],
                                add=True)

    def wait_scatter(y, sl):
        pltpu.make_async_copy(y.at[pl.ds(0, CHUNK)], rows[sl], sems[sl]).wait()

    def wait_gather(y, sl):
        pltpu.make_async_copy(y.at[pl.ds(0, CHUNK)], rows[sl], semg[sl]).wait()

    def block(y, u, pref_row, first):
        wait_bfetch(u)
        for k in range(AGB):
            sl = k % 2
            if not (first and k < 2):
                wait_scatter(y, sl)
            gather(y, u, k, sl)
            if not (first and k == 0):
                pu, pk = (u, k - 1) if k >= 1 else (1 - u, AGB - 1)
                wait_gather(y, 1 - sl)
                scatter(pu, pk, 1 - sl)
            if k == 1:
                bfetch(pref_row, 1 - u)

    for b, y in enumerate((y1, y2, y3)):
        row0 = b * NCHK + base
        bfetch(row0, 0)
        block(y, 0, row0 + AGB, True)
        block(y, 1, row0 + 2 * AGB, False)

        cap = row0 + (nblk - 2) * AGB
        def dbl(d, carry):
            r2 = row0 + 2 * d * AGB
            block(y, 0, r2 + AGB, False)
            block(y, 1, jnp.minimum(r2 + 2 * AGB, cap), False)
            return carry
        lax.fori_loop(1, nblk // 2, dbl, 0)
        wait_bfetch(0)
        wait_gather(y, 1)
        scatter(1, AGB - 1, 1)
        wait_scatter(y, 0)
        wait_scatter(y, 1)
        plsc.subcore_barrier()
        nq = RPT // CHUNK
        for q in range(nq):
            s_ = q % 2
            if q >= 2:
                wait_gather(y, s_)
            sl = pl.ds(sid * RPT + q * CHUNK, CHUNK)
            pltpu.sync_copy(acc.at[sl], rows[s_])
            pltpu.async_copy(rows[s_], out.at[b * NC + cid, sl], semg[s_])
            for w in range(CHUNK // ZBR):
                pltpu.async_copy(
                    zb, acc.at[pl.ds(sid * RPT + q * CHUNK + w * ZBR, ZBR)],
                    sems0)
        wait_gather(y, (nq - 2) % 2)
        wait_gather(y, (nq - 1) % 2)
        for _ in range(nq * (CHUNK // ZBR)):
            pltpu.make_async_copy(y.at[pl.ds(0, ZBR)], zb, sems0).wait()
        plsc.subcore_barrier()


def _run_agg(ed, y1, y2, y3):
    mesh = plsc.VectorSubcoreMesh(core_axis_name="c", subcore_axis_name="s")
    shp = jax.ShapeDtypeStruct((3 * NC, NP, D), jnp.float32)
    fn = pl.kernel(
        _agg_body,
        out_type=shp,
        mesh=mesh,
        scratch_types=[
            pltpu.VMEM((AGB, 2, CHUNK), jnp.int32),
            pltpu.VMEM((AGB, 2, CHUNK), jnp.int32),
            pltpu.VMEM((CHUNK, D), jnp.float32),
            pltpu.VMEM((CHUNK, D), jnp.float32),
            pltpu.VMEM((ZBR, D), jnp.float32),
            pltpu.VMEM_SHARED((NP, D), jnp.float32),
            pltpu.SemaphoreType.DMA,
            pltpu.SemaphoreType.DMA,
            pltpu.SemaphoreType.DMA,
            pltpu.SemaphoreType.DMA,
            pltpu.SemaphoreType.DMA,
            pltpu.SemaphoreType.DMA,
        ],
    )
    return fn(ed, y1, y2, y3)


def _scale_body(x1_ref, x2_ref, x3_ref, dp_ref, y1_ref, y2_ref, y3_ref):
    for b, (x_ref, y_ref) in enumerate(
            ((x1_ref, y1_ref), (x2_ref, y2_ref), (x3_ref, y3_ref))):
        deg = dp_ref[2 * b] + dp_ref[2 * b + 1]
        r = lax.rsqrt(jnp.maximum(deg[:N], 1.0))
        y_ref[:N] = x_ref[...] * r[:, None]
        y_ref[N:] = jnp.zeros((NP - N, D), jnp.float32)


def _run_scale(x1, x2, x3, deg_part):
    yshp = jax.ShapeDtypeStruct((NP, D), jnp.float32)
    return pl.pallas_call(
        _scale_body,
        out_shape=(yshp, yshp, yshp),
        compiler_params=pltpu.CompilerParams(vmem_limit_bytes=100 * 1024 * 1024),
    )(x1, x2, x3, deg_part)


def _mlp_apply(x, w_ref, b_ref, a0, a1, a2):
    h = jnp.dot(x, w_ref[0], preferred_element_type=jnp.float32) + b_ref[0]
    h = jnp.where(h >= 0, h, a0 * h)
    h = jnp.dot(h, w_ref[1], preferred_element_type=jnp.float32) + b_ref[1]
    h = jnp.where(h >= 0, h, a1 * h)
    h = jnp.dot(h, w_ref[2], preferred_element_type=jnp.float32) + b_ref[2]
    h = jnp.where(h >= 0, h, a2 * h)
    sc = jnp.dot(x, w_ref[3], preferred_element_type=jnp.float32) + b_ref[3]
    return h + sc


def _loss_body(tp_ref, dp_ref, ew_ref, eb_ref, batch_ref,
               lw_ref, lb_ref, gw_ref, gb_ref, al_ref, out_ref):
    t = tp_ref[0, 0] + tp_ref[0, 1]
    deg = dp_ref[0, 0] + dp_ref[0, 1]
    r = lax.rsqrt(jnp.maximum(deg, 1.0))
    agg = t * r[:, None]
    z = jnp.dot(agg, ew_ref[0], preferred_element_type=jnp.float32)
    z = jnp.maximum(z + eb_ref[0, 0], 0.0)
    zn = z[:N]

    bt = batch_ref[0, 0]
    gids = lax.broadcasted_iota(jnp.int32, (1, G), 1)
    pos = (bt[:, None] == gids).astype(jnp.float32)
    g = jax.lax.dot_general(pos, zn, (((0,), (0,)), ((), ())),
                            preferred_element_type=jnp.float32)

    a = al_ref[0]
    zl = _mlp_apply(zn, lw_ref, lb_ref, a[0], a[1], a[2])
    gl = _mlp_apply(g, gw_ref, gb_ref, a[3], a[4], a[5])

    res = jax.lax.dot_general(zl, gl, (((1,), (1,)), ((), ())),
                              preferred_element_type=jnp.float32)
    neg_res = -res
    sp = jnp.maximum(neg_res, 0.0) + jnp.log1p(jnp.exp(-jnp.abs(neg_res)))
    e_pos = _LOG2 - sp
    e_neg = sp + res - _LOG2
    npos = jnp.sum(pos)
    nneg = float(N * G) - npos
    e_pos_sum = jnp.sum(e_pos * pos)
    e_neg_sum = jnp.sum(e_neg) - jnp.sum(e_neg * pos)
    loss = 2.0 * (e_neg_sum / jnp.maximum(nneg, 1.0)
                  - e_pos_sum / jnp.maximum(npos, 1.0))
    out_ref[0] = jnp.full((8, 128), loss, jnp.float32)


def _run_loss(t_part, deg_part, enc_w, enc_b, batch, lw, lb, gw, gb, alphas):
    full = lambda *shape: pl.BlockSpec(shape, lambda b: (0,) * len(shape))
    return pl.pallas_call(
        _loss_body,
        grid=(3,),
        in_specs=[
            pl.BlockSpec((1, NC, NP, D), lambda b: (b, 0, 0, 0)),
            pl.BlockSpec((1, NC, NP), lambda b: (b, 0, 0)),
            pl.BlockSpec((1, D, D), lambda b: (b, 0, 0)),
            pl.BlockSpec((1, 1, D), lambda b: (b, 0, 0)),
            pl.BlockSpec((1, 1, N), lambda b: (b, 0, 0)),
            full(4, D, D),
            full(4, D),
            full(4, D, D),
            full(4, D),
            full(1, 8),
        ],
        out_specs=pl.BlockSpec((1, 8, 128), lambda b: (b, 0, 0)),
        out_shape=jax.ShapeDtypeStruct((3, 8, 128), jnp.float32),
        compiler_params=pltpu.CompilerParams(vmem_limit_bytes=100 * 1024 * 1024),
    )(t_part, deg_part, enc_w, enc_b, batch, lw, lb, gw, gb, alphas)


def kernel(x1, x2, x3, edge_index1, edge_index2, edge_index3,
           batch1, batch2, batch3, params):
    pad_fill = (N + (jnp.arange(EP - E, dtype=jnp.int32) % PAD_SPREAD))

    def pad_idx(ei):
        sr = jnp.concatenate([ei[0], pad_fill]).reshape(NCHK, 1, CHUNK)
        ds_ = jnp.concatenate([ei[1], pad_fill]).reshape(NCHK, 1, CHUNK)
        return jnp.concatenate([sr, ds_], axis=1)

    ed = jnp.concatenate([pad_idx(edge_index1), pad_idx(edge_index2),
                          pad_idx(edge_index3)], axis=0)

    deg_flat = _run_deg(ed)
    deg_part = deg_flat.reshape(3, NC, NP)

    y1, y2, y3 = _run_scale(x1, x2, x3, deg_flat)

    t_flat = _run_agg(ed, y1, y2, y3)
    t_part = t_flat.reshape(3, NC, NP, D)

    p = params
    enc_w = jnp.stack([p["enc1"]["W"], p["enc2"]["W"], p["enc3"]["W"]])
    enc_b = jnp.stack([p["enc1"]["b"], p["enc2"]["b"],
                       p["enc3"]["b"]])[:, None, :]
    batch = jnp.stack([batch1, batch2, batch3])[:, None, :]

    def mlp_params(mp):
        w = jnp.stack([mp["fc1"]["W"], mp["fc2"]["W"], mp["fc3"]["W"],
                       mp["sc"]["W"]])
        b = jnp.stack([mp["fc1"]["b"], mp["fc2"]["b"], mp["fc3"]["b"],
                       mp["sc"]["b"]])
        return w, b

    lw, lb = mlp_params(p["local"])
    gw, gb = mlp_params(p["global"])
    alphas = jnp.concatenate([p["local"]["a1"], p["local"]["a2"],
                              p["local"]["a3"], p["global"]["a1"],
                              p["global"]["a2"], p["global"]["a3"],
                              jnp.zeros((2,), jnp.float32)])[None, :]

    losses = _run_loss(t_part, deg_part, enc_w, enc_b, batch,
                       lw, lb, gw, gb, alphas)
    return losses[0, 0, 0] + losses[1, 0, 0] + losses[2, 0, 0]
